# baseline (speedup 1.0000x reference)
import jax
import jax.numpy as jnp
from jax import lax
from jax.experimental import pallas as pl
from jax.experimental.pallas import tpu as pltpu
from jax.experimental.pallas import tpu_sc as plsc

N_NODES = 10000
N_EDGES = 160000
D = 256
H = 128
NC = 2
NS = 16
N_PAD = 10240
EPW = N_PAD
E_PAD = NS * EPW
CH = 64
NPH = 2
CPP = EPW // CH // NPH
NBUF = 3
LAG = 2
NG = (CPP + LAG + NBUF - 1) // NBUF
WCH = CH
RPW = N_PAD // NS
NRCH = RPW // WCH


def _sc_body(feat0_hbm, feat1_hbm, src_hbm, dst_hbm, out0_hbm, out1_hbm,
             acc, sbuf, dbuf, rows, gsem, ssem):
    c = lax.axis_index("c")
    s = lax.axis_index("s")

    zbuf = rows[0]

    def _zero(i, _):
        zbuf[i // 8, pl.ds((i % 8) * 16, 16)] = jnp.zeros((16,), jnp.float32)
        return 0
    lax.fori_loop(0, WCH * 8, _zero, 0)
    row0 = s * RPW
    for k in range(NRCH):
        pltpu.sync_copy(zbuf, acc.at[pl.ds(row0 + k * WCH, WCH)])

    def _gather(r, b):
        @pl.when(c == 0)
        def _():
            pltpu.async_copy(feat0_hbm.at[sbuf.at[r]], rows[b], gsem[b])

        @pl.when(c == 1)
        def _():
            pltpu.async_copy(feat1_hbm.at[sbuf.at[r]], rows[b], gsem[b])

    def _wait_gather(b):
        pltpu.make_async_copy(feat0_hbm.at[sbuf.at[0]], rows[b], gsem[b]).wait()

    def _scatter(r, b):
        pltpu.async_copy(rows[b], acc.at[dbuf.at[r]], ssem[b], add=True)

    def _wait_scatter(b):
        pltpu.make_async_copy(rows[b], acc.at[dbuf.at[0]], ssem[b]).wait()

    first = True
    for ph in range(NPH):
        pltpu.sync_copy(src_hbm.at[s, ph], sbuf)
        pltpu.sync_copy(dst_hbm.at[s, ph], dbuf)
        if first:
            plsc.subcore_barrier()
            first = False

        def _group(g, _):
            for b in range(NBUF):
                r = g * NBUF + b

                @pl.when(jnp.logical_and(r >= NBUF, r < CPP))
                def _():
                    _wait_scatter(b)

                @pl.when(r < CPP)
                def _():
                    _gather(r, b)

                j = r - LAG
                bj = (b - LAG) % NBUF

                @pl.when(jnp.logical_and(j >= 0, j < CPP))
                def _():
                    _wait_gather(bj)
                    _scatter(j, bj)
            return 0

        lax.fori_loop(0, NG, _group, 0)
        for b in range(NBUF):
            _wait_scatter((CPP - 1 - b) % NBUF)

    plsc.subcore_barrier()

    wbuf = rows[1]
    for k in range(NRCH):
        off = row0 + k * WCH
        pltpu.sync_copy(acc.at[pl.ds(off, WCH)], wbuf)

        @pl.when(c == 0)
        def _():
            pltpu.sync_copy(wbuf, out0_hbm.at[pl.ds(off, WCH)])

        @pl.when(c == 1)
        def _():
            pltpu.sync_copy(wbuf, out1_hbm.at[pl.ds(off, WCH)])


_sc_segsum = pl.kernel(
    _sc_body,
    out_type=(jax.ShapeDtypeStruct((N_PAD, H), jnp.float32),
              jax.ShapeDtypeStruct((N_PAD, H), jnp.float32)),
    mesh=plsc.VectorSubcoreMesh(core_axis_name="c", subcore_axis_name="s"),
    scratch_types=[
        pltpu.VMEM_SHARED((N_PAD, H), jnp.float32),
        pltpu.VMEM((CPP, CH), jnp.int32),
        pltpu.VMEM((CPP, CH), jnp.int32),
        [pltpu.VMEM((CH, H), jnp.float32)] * NBUF,
        [pltpu.SemaphoreType.DMA] * NBUF,
        [pltpu.SemaphoreType.DMA] * NBUF,
    ],
)


def _mm_body2(a0_ref, a1_ref, wa_ref, wb_ref, o0_ref, o1_ref):
    h = jnp.dot(a0_ref[...], wa_ref[...], precision=lax.Precision.HIGHEST,
                preferred_element_type=jnp.float32)
    h += jnp.dot(a1_ref[...], wb_ref[...], precision=lax.Precision.HIGHEST,
                 preferred_element_type=jnp.float32)
    h = jnp.maximum(h, 0.0)
    o0_ref[...] = h[:, :H]
    o1_ref[...] = h[:, H:]


def _mm_body1(a0_ref, a1_ref, wa_ref, wb_ref, o_ref):
    h = jnp.dot(a0_ref[...], wa_ref[...], precision=lax.Precision.HIGHEST,
                preferred_element_type=jnp.float32)
    h += jnp.dot(a1_ref[...], wb_ref[...], precision=lax.Precision.HIGHEST,
                 preferred_element_type=jnp.float32)
    o_ref[...] = jnp.maximum(h, 0.0)


_BM = 1000


def _mm_relu(a0, a1, wa, wb, split):
    in_specs = [
        pl.BlockSpec((_BM, H), lambda i: (i, 0)),
        pl.BlockSpec((_BM, H), lambda i: (i, 0)),
        pl.BlockSpec((H, D), lambda i: (0, 0)),
        pl.BlockSpec((H, D), lambda i: (0, 0)),
    ]
    if split:
        return pl.pallas_call(
            _mm_body2,
            grid=(N_NODES // _BM,),
            in_specs=in_specs,
            out_specs=(pl.BlockSpec((_BM, H), lambda i: (i, 0)),
                       pl.BlockSpec((_BM, H), lambda i: (i, 0))),
            out_shape=(jax.ShapeDtypeStruct((N_NODES, H), jnp.float32),
                       jax.ShapeDtypeStruct((N_NODES, H), jnp.float32)),
        )(a0, a1, wa, wb)
    return pl.pallas_call(
        _mm_body1,
        grid=(N_NODES // _BM,),
        in_specs=in_specs,
        out_specs=pl.BlockSpec((_BM, D), lambda i: (i, 0)),
        out_shape=jax.ShapeDtypeStruct((N_NODES, D), jnp.float32),
    )(a0, a1, wa, wb)


def kernel(x, edge_index, batch, W1, W2):
    pad = E_PAD - N_EDGES
    src = jnp.concatenate([edge_index[0], jnp.zeros((pad,), jnp.int32)])
    dst = jnp.concatenate([edge_index[1],
                           jnp.full((pad,), N_NODES, jnp.int32)])
    src = src.reshape(NS, NPH, CPP, CH)
    dst = dst.reshape(NS, NPH, CPP, CH)

    f0, f1 = x[:, :H], x[:, H:]
    a0, a1 = _sc_segsum(f0, f1, src, dst)
    h0, h1 = _mm_relu(a0, a1, W1[:H], W1[H:], split=True)
    a0, a1 = _sc_segsum(h0, h1, src, dst)
    return _mm_relu(a0, a1, W2[:H], W2[H:], split=False)
